# Initial kernel scaffold; baseline (speedup 1.0000x reference)
import jax
import jax.numpy as jnp
from jax.experimental import pallas as pl


def kernel(x, edge_index, batch, edge_attr, params):
    raise NotImplementedError("write your pallas kernel here")


# speedup vs baseline: 3.2689x; 3.2689x over previous
import functools
import jax
import jax.numpy as jnp
from jax import lax
from jax.experimental import pallas as pl
from jax.experimental.pallas import tpu as pltpu
from jax.experimental.pallas import tpu_sc as plsc

N = 10000
E = 320000
D = 128
DE = 16
H = 128
OUT = 128
L = 2

NC = 2
NS = 16
NW = NC * NS
C = 128
N_PAD = 10112
TILE_ROWS = N_PAD // NS
E_PER_TEC = 10112
NCHUNK = E_PER_TEC // C
E_PAD = E_PER_TEC * NW
DEGW = 8


def _mlp_call(x, W1, b1, W2, b2, block):
    rows = x.shape[0]
    k = x.shape[1]
    out_d = W2.shape[1]

    def body(x_ref, w1_ref, b1_ref, w2_ref, b2_ref, o_ref):
        h = jnp.maximum(
            jnp.dot(x_ref[...], w1_ref[...], preferred_element_type=jnp.float32)
            + b1_ref[...], 0.0)
        o_ref[...] = jnp.dot(h, w2_ref[...],
                             preferred_element_type=jnp.float32) + b2_ref[...]

    return pl.pallas_call(
        body,
        grid=(rows // block,),
        in_specs=[
            pl.BlockSpec((block, k), lambda i: (i, 0)),
            pl.BlockSpec(W1.shape, lambda i: (0, 0)),
            pl.BlockSpec((1, W1.shape[1]), lambda i: (0, 0)),
            pl.BlockSpec(W2.shape, lambda i: (0, 0)),
            pl.BlockSpec((1, out_d), lambda i: (0, 0)),
        ],
        out_specs=pl.BlockSpec((block, out_d), lambda i: (i, 0)),
        out_shape=jax.ShapeDtypeStruct((rows, out_d), jnp.float32),
    )(x, W1, b1[None], W2, b2[None])


def _ab_call(h, W1i, W1j, b1, block=1264):
    rows = h.shape[0]

    def body(h_ref, wi_ref, wj_ref, b1_ref, a_ref, b_ref):
        hb = h_ref[...]
        a_ref[...] = jnp.dot(hb, wi_ref[...],
                             preferred_element_type=jnp.float32) + b1_ref[...]
        b_ref[...] = jnp.dot(hb, wj_ref[...],
                             preferred_element_type=jnp.float32)

    return pl.pallas_call(
        body,
        grid=(rows // block,),
        in_specs=[
            pl.BlockSpec((block, H), lambda i: (i, 0)),
            pl.BlockSpec((H, H), lambda i: (0, 0)),
            pl.BlockSpec((H, H), lambda i: (0, 0)),
            pl.BlockSpec((1, H), lambda i: (0, 0)),
        ],
        out_specs=[
            pl.BlockSpec((block, H), lambda i: (i, 0)),
            pl.BlockSpec((block, H), lambda i: (i, 0)),
        ],
        out_shape=[
            jax.ShapeDtypeStruct((rows, H), jnp.float32),
            jax.ShapeDtypeStruct((rows, H), jnp.float32),
        ],
    )(h, W1i, W1j, b1[None])


def _ce_call(ea, W1e, block=2048):
    rows = ea.shape[0]

    def body(e_ref, w_ref, o_ref):
        o_ref[...] = jnp.dot(e_ref[...], w_ref[...],
                             preferred_element_type=jnp.float32)

    return pl.pallas_call(
        body,
        grid=(rows // block,),
        in_specs=[
            pl.BlockSpec((block, DE), lambda i: (i, 0)),
            pl.BlockSpec((DE, H), lambda i: (0, 0)),
        ],
        out_specs=pl.BlockSpec((block, H), lambda i: (i, 0)),
        out_shape=jax.ShapeDtypeStruct((rows, H), jnp.float32),
    )(ea, W1e)


def _update_call(Sp, degt, h, W2, b2, U1t, U1b, ub1, U2, ub2, block=1264):
    rows = h.shape[0]

    def body(sp_ref, dg_ref, h_ref, w2_ref, b2_ref, u1t_ref, u1b_ref,
             ub1_ref, u2_ref, ub2_ref, o_ref):
        S = sp_ref[0] + sp_ref[1]
        deg = dg_ref[0, :, :1] + dg_ref[1, :, :1]
        agg = jnp.dot(S, w2_ref[...],
                      preferred_element_type=jnp.float32) + deg * b2_ref[...]
        t = jnp.maximum(
            jnp.dot(h_ref[...], u1t_ref[...], preferred_element_type=jnp.float32)
            + jnp.dot(agg, u1b_ref[...], preferred_element_type=jnp.float32)
            + ub1_ref[...], 0.0)
        o_ref[...] = jnp.dot(t, u2_ref[...],
                             preferred_element_type=jnp.float32) + ub2_ref[...]

    full = lambda shape: pl.BlockSpec(shape, lambda i: tuple(0 for _ in shape))
    return pl.pallas_call(
        body,
        grid=(rows // block,),
        in_specs=[
            pl.BlockSpec((NC, block, H), lambda i: (0, i, 0)),
            pl.BlockSpec((NC, block, DEGW), lambda i: (0, i, 0)),
            pl.BlockSpec((block, H), lambda i: (i, 0)),
            full((H, H)), full((1, H)), full((H, H)), full((H, H)),
            full((1, H)), full((H, H)), full((1, H)),
        ],
        out_specs=pl.BlockSpec((block, H), lambda i: (i, 0)),
        out_shape=jax.ShapeDtypeStruct((rows, H), jnp.float32),
    )(Sp, degt, h, W2, b2[None], U1t, U1b, ub1[None], U2, ub2[None])


def _make_sc_kernel():
    mesh = plsc.VectorSubcoreMesh(core_axis_name="c", subcore_axis_name="s")

    def body(A_hbm, B_hbm, Ce_hbm, dst_hbm, src_hbm, z_hbm,
             S_out, dst_v, src_v, bufA, bufB, bufC, S_sh, sem_a, sem_b):
        c = lax.axis_index("c")
        s = lax.axis_index("s")
        wid = s * NC + c
        row_base = s * TILE_ROWS

        pltpu.sync_copy(z_hbm, S_sh.at[pl.ds(row_base, TILE_ROWS)])
        plsc.subcore_barrier()

        def chunk_body(i, carry):
            ebase = wid * E_PER_TEC + i * C
            pltpu.sync_copy(dst_hbm.at[pl.ds(ebase, C)], dst_v)
            pltpu.sync_copy(src_hbm.at[pl.ds(ebase, C)], src_v)
            ca = pltpu.async_copy(A_hbm.at[dst_v], bufA, sem_a)
            cb = pltpu.async_copy(B_hbm.at[src_v], bufB, sem_b)
            pltpu.sync_copy(Ce_hbm.at[pl.ds(ebase, C)], bufC)
            ca.wait()
            cb.wait()

            def row_body(r, carry2):
                for j in range(H // 16):
                    sl = pl.ds(j * 16, 16)
                    v = bufA[r, sl] + bufB[r, sl] + bufC[r, sl]
                    bufC[r, sl] = jnp.maximum(v, 0.0)
                return carry2

            lax.fori_loop(0, C, row_body, 0)
            pltpu.sync_copy(bufC, S_sh.at[dst_v], add=True)
            return carry

        lax.fori_loop(0, NCHUNK, chunk_body, 0)
        plsc.subcore_barrier()

        sl = pl.ds(row_base, TILE_ROWS)
        pltpu.sync_copy(S_sh.at[sl], S_out.at[c, sl])

    return pl.kernel(
        body,
        out_type=jax.ShapeDtypeStruct((NC, N_PAD, H), jnp.float32),
        mesh=mesh,
        scratch_types=(
            pltpu.VMEM((C,), jnp.int32),
            pltpu.VMEM((C,), jnp.int32),
            pltpu.VMEM((C, H), jnp.float32),
            pltpu.VMEM((C, H), jnp.float32),
            pltpu.VMEM((C, H), jnp.float32),
            pltpu.VMEM_SHARED((N_PAD, H), jnp.float32),
            pltpu.SemaphoreType.DMA,
            pltpu.SemaphoreType.DMA,
        ))


def _make_deg_kernel():
    mesh = plsc.VectorSubcoreMesh(core_axis_name="c", subcore_axis_name="s")

    def body(dst_hbm, z8_hbm, ones_hbm,
             deg_out, dst_v, ones_v, deg_sh):
        c = lax.axis_index("c")
        s = lax.axis_index("s")
        wid = s * NC + c
        row_base = s * TILE_ROWS

        pltpu.sync_copy(z8_hbm, deg_sh.at[pl.ds(row_base, TILE_ROWS)])
        pltpu.sync_copy(ones_hbm, ones_v)
        plsc.subcore_barrier()

        def chunk_body(i, carry):
            ebase = wid * E_PER_TEC + i * C
            pltpu.sync_copy(dst_hbm.at[pl.ds(ebase, C)], dst_v)
            pltpu.sync_copy(ones_v, deg_sh.at[dst_v], add=True)
            return carry

        lax.fori_loop(0, NCHUNK, chunk_body, 0)
        plsc.subcore_barrier()

        sl = pl.ds(row_base, TILE_ROWS)
        pltpu.sync_copy(deg_sh.at[sl], deg_out.at[c, sl])

    return pl.kernel(
        body,
        out_type=jax.ShapeDtypeStruct((NC, N_PAD, DEGW), jnp.float32),
        mesh=mesh,
        scratch_types=(
            pltpu.VMEM((C,), jnp.int32),
            pltpu.VMEM((C, DEGW), jnp.float32),
            pltpu.VMEM_SHARED((N_PAD, DEGW), jnp.float32),
        ))


_sc_edge = _make_sc_kernel()
_sc_deg = _make_deg_kernel()


def kernel(x, edge_index, batch, edge_attr, params):
    p = params
    src = edge_index[0]
    dst = edge_index[1]
    dst_p = jnp.concatenate(
        [dst, jnp.full((E_PAD - E,), N, dtype=jnp.int32)])
    src_p = jnp.concatenate(
        [src, jnp.zeros((E_PAD - E,), dtype=jnp.int32)])
    ea_p = jnp.pad(edge_attr, ((0, E_PAD - E), (0, 0)))
    x_p = jnp.pad(x, ((0, N_PAD - N), (0, 0)))

    zeros = jnp.zeros((TILE_ROWS, H), jnp.float32)
    zeros8 = jnp.zeros((TILE_ROWS, DEGW), jnp.float32)
    ones = jnp.ones((C, DEGW), jnp.float32)

    h = _mlp_call(x_p, p['emb_W1'], p['emb_b1'], p['emb_W2'], p['emb_b2'],
                  block=1264)

    degt = _sc_deg(dst_p, zeros8, ones)
    for l in range(L):
        W1 = p[f'l{l}_msg_W1']
        A, B = _ab_call(h, W1[:H], W1[H:2 * H], p[f'l{l}_msg_b1'])
        Ce = _ce_call(ea_p, W1[2 * H:])
        Sp = _sc_edge(A, B, Ce, dst_p, src_p, zeros)
        U1 = p[f'l{l}_upd_W1']
        h = _update_call(Sp, degt, h, p[f'l{l}_msg_W2'], p[f'l{l}_msg_b2'],
                         U1[:H], U1[H:], p[f'l{l}_upd_b1'],
                         p[f'l{l}_upd_W2'], p[f'l{l}_upd_b2'])

    out = _mlp_call(h, p['head_W1'], p['head_b1'], p['head_W2'],
                    p['head_b2'], block=1264)
    return out[:N]
